# Initial kernel scaffold; baseline (speedup 1.0000x reference)
import jax
import jax.numpy as jnp
from jax.experimental import pallas as pl


def kernel(x, edge_index, num_nodes, adaptive_weight):
    raise NotImplementedError("write your pallas kernel here")


# speedup vs baseline: 16.4441x; 16.4441x over previous
import functools

import jax
import jax.numpy as jnp
from jax import lax
from jax.experimental import pallas as pl
from jax.experimental.pallas import tpu as pltpu
from jax.experimental.pallas import tpu_sc as plsc

NC = 2
NS = 16
LANES = 16
CHUNK = 128


def _sc_mesh():
    return plsc.VectorSubcoreMesh(core_axis_name="c", subcore_axis_name="s")


def _deg_hist(E, NPAD):
    nchunk = E // CHUNK
    nw = NC * NS
    full, tail = nchunk // nw, nchunk % nw
    seg = NPAD // NS

    @functools.partial(
        pl.kernel,
        out_type=jax.ShapeDtypeStruct((NC * NPAD,), jnp.float32),
        mesh=_sc_mesh(),
        scratch_types=[
            pltpu.VMEM((CHUNK,), jnp.int32),
            pltpu.VMEM((CHUNK,), jnp.float32),
            pltpu.VMEM((seg,), jnp.float32),
            pltpu.VMEM_SHARED((NPAD,), jnp.float32),
        ],
    )
    def k(row_hbm, out_hbm, idx_v, ones_v, z_v, deg_sh):
        c = lax.axis_index("c")
        s = lax.axis_index("s")
        wid = s * NC + c

        for j in range(CHUNK // LANES):
            ones_v[pl.ds(j * LANES, LANES)] = jnp.ones((LANES,), jnp.float32)

        @pl.loop(0, seg // LANES)
        def _(j):
            z_v[pl.ds(j * LANES, LANES)] = jnp.zeros((LANES,), jnp.float32)

        pltpu.sync_copy(z_v, deg_sh.at[pl.ds(s * seg, seg)])
        plsc.subcore_barrier()

        def do_chunk(chunkid):
            base = chunkid * CHUNK
            pltpu.sync_copy(row_hbm.at[pl.ds(base, CHUNK)], idx_v)
            pltpu.sync_copy(ones_v, deg_sh.at[idx_v], add=True)

        @pl.loop(0, full)
        def _(i):
            do_chunk(i * nw + wid)

        @pl.when(wid < tail)
        def _():
            do_chunk(full * nw + wid)

        plsc.subcore_barrier()
        pltpu.sync_copy(deg_sh.at[pl.ds(s * seg, seg)], z_v)
        pltpu.sync_copy(z_v, out_hbm.at[pl.ds(c * NPAD + s * seg, seg)])

    return k


def _edge_scatter(E, NPAD, D):
    nchunk = E // CHUNK
    nw = NC * NS
    full, tail = nchunk // nw, nchunk % nw
    rps = NPAD // NS

    @functools.partial(
        pl.kernel,
        out_type=jax.ShapeDtypeStruct((NC, NPAD, D), jnp.float32),
        mesh=_sc_mesh(),
        scratch_types=[
            pltpu.VMEM((CHUNK,), jnp.int32),
            pltpu.VMEM((CHUNK,), jnp.int32),
            pltpu.VMEM((CHUNK, D), jnp.float32),
            pltpu.VMEM_SHARED((NPAD, D), jnp.float32),
            pltpu.SemaphoreType.DMA,
        ],
    )
    def k(row_hbm, col_hbm, y_hbm, out_hbm, ridx, cidx, rows, acc_sh, sem):
        c = lax.axis_index("c")
        s = lax.axis_index("s")
        wid = s * NC + c
        nfull, remr = rps // CHUNK, rps % CHUNK

        @pl.loop(0, CHUNK)
        def _(i):
            for j in range(D // LANES):
                rows[i, pl.ds(j * LANES, LANES)] = jnp.zeros((LANES,), jnp.float32)

        for t in range(nfull):
            pltpu.sync_copy(rows, acc_sh.at[pl.ds(s * rps + t * CHUNK, CHUNK)])
        if remr:
            pltpu.sync_copy(
                rows.at[pl.ds(0, remr)],
                acc_sh.at[pl.ds(s * rps + nfull * CHUNK, remr)],
            )
        plsc.subcore_barrier()

        def do_chunk(chunkid):
            base = chunkid * CHUNK
            pltpu.sync_copy(row_hbm.at[pl.ds(base, CHUNK)], ridx)
            pltpu.sync_copy(col_hbm.at[pl.ds(base, CHUNK)], cidx)
            pltpu.async_copy(y_hbm.at[ridx], rows, sem).wait()
            pltpu.sync_copy(rows, acc_sh.at[cidx], add=True)

        @pl.loop(0, full)
        def _(i):
            do_chunk(i * nw + wid)

        @pl.when(wid < tail)
        def _():
            do_chunk(full * nw + wid)

        plsc.subcore_barrier()
        for t in range(nfull):
            sl = pl.ds(s * rps + t * CHUNK, CHUNK)
            pltpu.sync_copy(acc_sh.at[sl], rows)
            pltpu.sync_copy(rows, out_hbm.at[c, sl])
        if remr:
            sl = pl.ds(s * rps + nfull * CHUNK, remr)
            pltpu.sync_copy(acc_sh.at[sl], rows.at[pl.ds(0, remr)])
            pltpu.sync_copy(rows.at[pl.ds(0, remr)], out_hbm.at[c, sl])

    return k


def _dis_from_deg(deg_blk):
    deg = deg_blk[:, 0:1] + deg_blk[:, 1:2]
    return jnp.where(deg > 0, lax.rsqrt(deg), 0.0)


def _y_body(deg_ref, x_ref, y_ref):
    y_ref[...] = jnp.maximum(x_ref[...], 0.0) * _dis_from_deg(deg_ref[...])


def _out_body(aw_ref, deg_ref, x_ref, acc_ref, o_ref):
    dis = _dis_from_deg(deg_ref[...])
    o_ref[...] = x_ref[...] + aw_ref[0] * dis * (acc_ref[0] + acc_ref[1])


def kernel(x, edge_index, num_nodes, adaptive_weight):
    N, D = x.shape
    E = edge_index.shape[1]
    row = edge_index[0]
    col = edge_index[1]
    aw = jnp.reshape(adaptive_weight, (1,)).astype(jnp.float32)

    npad = -(-N // (NS * 8)) * (NS * 8)
    deg2 = _deg_hist(E, npad)(row).reshape(NC, npad)
    deg_t = deg2.T[:N]

    R = 400
    grid = N // R
    y = pl.pallas_call(
        _y_body,
        grid=(grid,),
        in_specs=[
            pl.BlockSpec((R, NC), lambda i: (i, 0)),
            pl.BlockSpec((R, D), lambda i: (i, 0)),
        ],
        out_specs=pl.BlockSpec((R, D), lambda i: (i, 0)),
        out_shape=jax.ShapeDtypeStruct((N, D), jnp.float32),
    )(deg_t, x)

    acc2 = _edge_scatter(E, npad, D)(row, col, y)

    out = pl.pallas_call(
        _out_body,
        grid=(grid,),
        in_specs=[
            pl.BlockSpec(memory_space=pltpu.SMEM),
            pl.BlockSpec((R, NC), lambda i: (i, 0)),
            pl.BlockSpec((R, D), lambda i: (i, 0)),
            pl.BlockSpec((NC, R, D), lambda i: (0, i, 0)),
        ],
        out_specs=pl.BlockSpec((R, D), lambda i: (i, 0)),
        out_shape=jax.ShapeDtypeStruct((N, D), jnp.float32),
    )(aw, deg_t, x, acc2)
    return out
